# baseline (speedup 1.0000x reference)
import functools

import jax
import jax.numpy as jnp
from jax import lax
from jax.experimental import pallas as pl
from jax.experimental.pallas import tpu as pltpu
from jax.experimental.pallas import tpu_sc as plsc

DIM = 64
BATCH = 16384
ROWS2 = 500000
W2 = 2 * DIM

_info = plsc.get_sparse_core_info()
_NC = _info.num_cores
_NS = _info.num_subcores
_NW = _NC * _NS
_BPW = BATCH // _NW
_CH = 128
_NCH = _BPW // _CH
_NSLOT = 2

_mesh = plsc.VectorSubcoreMesh(core_axis_name="c", subcore_axis_name="s")


@functools.partial(
    pl.kernel,
    mesh=_mesh,
    out_type=(
        jax.ShapeDtypeStruct((BATCH, W2), jnp.float32),
        jax.ShapeDtypeStruct((BATCH, W2), jnp.float32),
    ),
    scratch_types=[
        pltpu.VMEM((_BPW,), jnp.int32),
        pltpu.VMEM((_NSLOT, _CH, W2), jnp.float32),
        pltpu.VMEM((_NSLOT, _CH, W2), jnp.float32),
        pltpu.SemaphoreType.DMA,
        pltpu.SemaphoreType.DMA,
        pltpu.SemaphoreType.DMA,
        pltpu.SemaphoreType.DMA,
    ],
)
def _dual_gather(edge_hbm, node_hbm, idx_hbm, out_e, out_n,
                 idx_v, ebuf, nbuf, sem_ge, sem_gn, sem_we, sem_wn):
    wid = lax.axis_index("s") * _NC + lax.axis_index("c")
    base = wid * _BPW
    pltpu.sync_copy(idx_hbm.at[pl.ds(base, _BPW)], idx_v)

    def gather_start(j):
        s = j % _NSLOT
        sl = pl.ds(j * _CH, _CH)
        ge = pltpu.async_copy(edge_hbm.at[idx_v.at[sl]], ebuf.at[s], sem_ge)
        gn = pltpu.async_copy(node_hbm.at[idx_v.at[sl]], nbuf.at[s], sem_gn)
        return ge, gn

    def write_start(j):
        s = j % _NSLOT
        out_sl = pl.ds(base + j * _CH, _CH)
        we = pltpu.async_copy(ebuf.at[s], out_e.at[out_sl], sem_we)
        wn = pltpu.async_copy(nbuf.at[s], out_n.at[out_sl], sem_wn)
        return we, wn

    gathers = [None] * _NCH
    writes = [None] * _NCH
    gathers[0] = gather_start(0)
    gathers[1] = gather_start(1)
    for j in range(_NCH):
        ge, gn = gathers[j]
        ge.wait()
        gn.wait()
        writes[j] = write_start(j)
        if j + _NSLOT < _NCH:
            we, wn = writes[j]
            we.wait()
            wn.wait()
            gathers[j + _NSLOT] = gather_start(j + _NSLOT)
    for j in range(_NCH - _NSLOT, _NCH):
        we, wn = writes[j]
        we.wait()
        wn.wait()


def kernel(edge_emb, node_emb, feat):
    e2 = edge_emb.reshape(ROWS2, W2)
    n2 = node_emb.reshape(ROWS2, W2)
    idx2 = lax.shift_right_logical(feat, 1)
    oe2, on2 = _dual_gather(e2, n2, idx2)
    odd = jnp.equal(jnp.bitwise_and(feat, 1), 1)[:, None]
    out_e = jnp.where(odd, oe2[:, DIM:], oe2[:, :DIM])
    out_n = jnp.where(odd, on2[:, DIM:], on2[:, :DIM])
    return out_e, out_n
